# baseline (speedup 1.0000x reference)
import jax
import jax.numpy as jnp
from jax.experimental import pallas as pl
from jax.experimental.pallas import tpu as pltpu
from jax.experimental.pallas import tpu_sc as plsc


_ROWS_PER_TILE = 2048


def _tc_argmin_body(z_ref, e_ref, idx_ref):
    z = z_ref[...]
    e = e_ref[...]
    r = z.shape[0]
    k = e.shape[0]
    zn = jnp.sum(z * z, axis=1, keepdims=True)
    en = jnp.sum(e * e, axis=1)
    p2 = jax.lax.dot_general(
        z * (-2.0), e, (((1,), (1,)), ((), ())),
        preferred_element_type=jnp.float32,
    )
    lane = jax.lax.broadcasted_iota(jnp.int32, (r, 128), 1)
    best_v = (zn + p2[:, 0:128]) + en[None, 0:128]
    best_i = lane
    for c in range(1, k // 128):
        dk = (zn + p2[:, c * 128:(c + 1) * 128]) + en[None, c * 128:(c + 1) * 128]
        m = dk < best_v
        best_v = jnp.where(m, dk, best_v)
        best_i = jnp.where(m, lane + (c * 128), best_i)
    rowmin = jnp.min(best_v, axis=1, keepdims=True)
    big = jnp.int32(2**30)
    sel = jnp.where(best_v == rowmin, best_i, big)
    idx = jnp.min(sel, axis=1)
    idx_ref[0, 0, :] = idx


def _argmin_indices(z_flat, embedding):
    n, d = z_flat.shape
    k = embedding.shape[0]
    r = _ROWS_PER_TILE
    t = n // r
    out = pl.pallas_call(
        _tc_argmin_body,
        grid=(t,),
        in_specs=[
            pl.BlockSpec((r, d), lambda i: (i, 0)),
            pl.BlockSpec((k, d), lambda i: (0, 0)),
        ],
        out_specs=pl.BlockSpec((1, 1, r), lambda i: (i, 0, 0)),
        out_shape=jax.ShapeDtypeStruct((t, 1, r), jnp.int32),
    )(z_flat, embedding)
    return out.reshape(n)


_GATHER_WINDOW = 128


def _sc_gather(embedding, indices):
    n = indices.shape[0]
    k, d = embedding.shape
    dp = 128
    e_pad = jnp.pad(embedding, ((0, 0), (0, dp - d)))
    w = _GATHER_WINDOW
    idx2 = indices.reshape(1, n)
    mesh = plsc.VectorSubcoreMesh(core_axis_name="core",
                                  subcore_axis_name="subcore")

    @pl.kernel(out_type=jax.ShapeDtypeStruct((n, dp), embedding.dtype),
               mesh=mesh)
    def gather_kernel(e_hbm, i_hbm, o_hbm):
        def body(i_vmem, o_vmem):
            pltpu.sync_copy(e_hbm.at[i_vmem.at[0]], o_vmem)

        pltpu.emit_pipeline(
            body,
            grid=(n // w,),
            in_specs=[pl.BlockSpec((1, w), index_map=lambda i: (0, i))],
            out_specs=[pl.BlockSpec((w, dp), index_map=lambda i: (i, 0))],
            core_axis_name=("core", "subcore"),
            dimension_semantics=(pltpu.PARALLEL,),
        )(i_hbm, o_hbm)

    return gather_kernel(e_pad, idx2)[:, :d]


def kernel(z, embedding):
    d = embedding.shape[1]
    z_flat = z.reshape(-1, d)
    indices = _argmin_indices(z_flat, embedding)
    z_q = _sc_gather(embedding, indices)
    return z_q.reshape(z.shape), indices


# speedup vs baseline: 1.4380x; 1.4380x over previous
import jax
import jax.numpy as jnp
from jax.experimental import pallas as pl
from jax.experimental.pallas import tpu as pltpu
from jax.experimental.pallas import tpu_sc as plsc


_ROWS_PER_TILE = 2048


def _tc_argmin_body(z_ref, e_ref, idx_ref):
    z = z_ref[...]
    e = e_ref[...]
    k = e.shape[0]
    en = jnp.sum(e * e, axis=1, keepdims=True)
    p2 = jax.lax.dot_general(
        e * (-2.0), z, (((1,), (1,)), ((), ())),
        preferred_element_type=jnp.float32,
    )
    sub = jax.lax.broadcasted_iota(jnp.int32, (8, p2.shape[1]), 0)
    best_v = en[0:8] + p2[0:8, :]
    best_i = sub
    for c in range(1, k // 8):
        vs = en[8 * c:8 * (c + 1)] + p2[8 * c:8 * (c + 1), :]
        m = vs < best_v
        best_v = jnp.where(m, vs, best_v)
        best_i = jnp.where(m, sub + (8 * c), best_i)
    minv = jnp.min(best_v, axis=0, keepdims=True)
    sel = jnp.where(best_v == minv, best_i, jnp.int32(2**30))
    idx = jnp.min(sel, axis=0)
    idx_ref[0, 0, :] = idx


def _argmin_indices(z_flat, embedding):
    n, d = z_flat.shape
    k = embedding.shape[0]
    r = _ROWS_PER_TILE
    t = n // r
    out = pl.pallas_call(
        _tc_argmin_body,
        grid=(t,),
        in_specs=[
            pl.BlockSpec((r, d), lambda i: (i, 0)),
            pl.BlockSpec((k, d), lambda i: (0, 0)),
        ],
        out_specs=pl.BlockSpec((1, 1, r), lambda i: (i, 0, 0)),
        out_shape=jax.ShapeDtypeStruct((t, 1, r), jnp.int32),
    )(z_flat, embedding)
    return out.reshape(n)


_GATHER_WINDOW = 128


def _sc_gather(embedding, indices):
    n = indices.shape[0]
    k, d = embedding.shape
    dp = 128
    e_pad = jnp.pad(embedding, ((0, 0), (0, dp - d)))
    w = _GATHER_WINDOW
    idx2 = indices.reshape(1, n)
    mesh = plsc.VectorSubcoreMesh(core_axis_name="core",
                                  subcore_axis_name="subcore")

    @pl.kernel(out_type=jax.ShapeDtypeStruct((n, dp), embedding.dtype),
               mesh=mesh)
    def gather_kernel(e_hbm, i_hbm, o_hbm):
        def body(i_vmem, o_vmem):
            pltpu.sync_copy(e_hbm.at[i_vmem.at[0]], o_vmem)

        pltpu.emit_pipeline(
            body,
            grid=(n // w,),
            in_specs=[pl.BlockSpec((1, w), index_map=lambda i: (0, i))],
            out_specs=[pl.BlockSpec((w, dp), index_map=lambda i: (i, 0))],
            core_axis_name=("core", "subcore"),
            dimension_semantics=(pltpu.PARALLEL,),
        )(i_hbm, o_hbm)

    return gather_kernel(e_pad, idx2)[:, :d]


def kernel(z, embedding):
    d = embedding.shape[1]
    z_flat = z.reshape(-1, d)
    indices = _argmin_indices(z_flat, embedding)
    z_q = _sc_gather(embedding, indices)
    return z_q.reshape(z.shape), indices
